# baseline (speedup 1.0000x reference)
import functools

import jax
import jax.numpy as jnp
from jax import lax
from jax.experimental import pallas as pl
from jax.experimental.pallas import tpu as pltpu
from jax.experimental.pallas import tpu_sc as plsc

_INFO = plsc.get_sparse_core_info()
_NC, _NS, _L = _INFO.num_cores, _INFO.num_subcores, _INFO.num_lanes
_NW = _NC * _NS

_BATCH = 4096
_SEQ = 50
_D = 768
_H = 256
_NE = 28
_HV = _H // 16

_B_PER_W = _BATCH // _NW
_ROWS_PER_CHUNK = 2
_IDS_REAL = _ROWS_PER_CHUNK * _SEQ
_IDS_PAD = 104
_CHUNKS = _B_PER_W // _ROWS_PER_CHUNK

_BM = 512
_MB = -(-30522 // _BM)


def _t1_body(e_ref, w_ref, o_ref):
    o_ref[...] = jnp.dot(e_ref[...], w_ref[...],
                         preferred_element_type=jnp.float32)


def _pool_body(ids_hbm, t1_hbm, out_hbm, idx_v, rows_v, acc_v, sem0, sem1):
    wid = lax.axis_index("s") * _NC + lax.axis_index("c")
    pltpu.sync_copy(ids_hbm.at[wid], idx_v)
    sems = (sem0, sem1)

    for b in range(2):
        pltpu.async_copy(t1_hbm.at[idx_v.at[b]], rows_v.at[b], sems[b])

    def pair_body(c2, carry):
        for b in range(2):
            c = 2 * c2 + b
            pltpu.make_async_copy(t1_hbm.at[idx_v.at[c]],
                                  rows_v.at[b], sems[b]).wait()
            for k in range(_ROWS_PER_CHUNK):
                def seq_body(s, acc):
                    r = k * _SEQ + s
                    return tuple(acc[j] + rows_v[b, r, pl.ds(16 * j, 16)]
                                 for j in range(_HV))
                acc = lax.fori_loop(
                    0, _SEQ, seq_body,
                    tuple(jnp.zeros((16,), jnp.float32) for _ in range(_HV)))
                row = c * _ROWS_PER_CHUNK + k
                for j in range(_HV):
                    acc_v[row, pl.ds(16 * j, 16)] = acc[j]

            @pl.when(c + 2 < _CHUNKS)
            def _prefetch():
                pltpu.async_copy(t1_hbm.at[idx_v.at[c + 2]],
                                 rows_v.at[b], sems[b])
        return carry

    lax.fori_loop(0, _CHUNKS // 2, pair_body, 0)
    pltpu.sync_copy(acc_v, out_hbm.at[pl.ds(wid * _B_PER_W, _B_PER_W)])


def _mlp_body(p_ref, b1_ref, w2_ref, b2_ref, o_ref):
    h = jnp.maximum(p_ref[...] * (1.0 / _SEQ) + b1_ref[...], 0.0)
    o_ref[...] = jnp.dot(h, w2_ref[...],
                         preferred_element_type=jnp.float32) + b2_ref[...]


def kernel(input_ids, emb_table, W1, b1, W2, b2):
    T1 = pl.pallas_call(
        _t1_body,
        grid=(_MB,),
        in_specs=[pl.BlockSpec((_BM, _D), lambda i: (i, 0)),
                  pl.BlockSpec((_D, _H), lambda i: (0, 0))],
        out_specs=pl.BlockSpec((_BM, _H), lambda i: (i, 0)),
        out_shape=jax.ShapeDtypeStruct((_MB * _BM, _H), jnp.float32),
    )(emb_table, W1)

    ids = input_ids.astype(jnp.int32).reshape(_NW, _CHUNKS, _IDS_REAL)
    ids = jnp.pad(ids, ((0, 0), (0, 0), (0, _IDS_PAD - _IDS_REAL)))

    pool = functools.partial(
        pl.kernel,
        mesh=plsc.VectorSubcoreMesh(core_axis_name="c", subcore_axis_name="s"),
        out_type=jax.ShapeDtypeStruct((_BATCH, _H), jnp.float32),
        scratch_types=[
            pltpu.VMEM((_CHUNKS, _IDS_PAD), jnp.int32),
            pltpu.VMEM((2, _IDS_PAD, _H), jnp.float32),
            pltpu.VMEM((_B_PER_W, _H), jnp.float32),
            pltpu.SemaphoreType.DMA,
            pltpu.SemaphoreType.DMA,
        ],
    )(_pool_body)
    P = pool(ids, T1)

    return pl.pallas_call(
        _mlp_body,
        grid=(8,),
        in_specs=[pl.BlockSpec((_BATCH // 8, _H), lambda i: (i, 0)),
                  pl.BlockSpec((1, _H), lambda i: (0, 0)),
                  pl.BlockSpec((_H, _NE), lambda i: (0, 0)),
                  pl.BlockSpec((1, _NE), lambda i: (0, 0))],
        out_specs=pl.BlockSpec((_BATCH // 8, _NE), lambda i: (i, 0)),
        out_shape=jax.ShapeDtypeStruct((_BATCH, _NE), jnp.float32),
    )(P, b1[None], W2, b2[None])


# speedup vs baseline: 2.6408x; 2.6408x over previous
import functools

import jax
import jax.numpy as jnp
from jax import lax
from jax.experimental import pallas as pl
from jax.experimental.pallas import tpu as pltpu
from jax.experimental.pallas import tpu_sc as plsc

_INFO = plsc.get_sparse_core_info()
_NC, _NS, _L = _INFO.num_cores, _INFO.num_subcores, _INFO.num_lanes
_NW = _NC * _NS

_BATCH = 4096
_SEQ = 50
_D = 768
_H = 256
_NE = 28
_HV = _H // 16

_B_PER_W = _BATCH // _NW
_ROWS_PER_CHUNK = 2
_IDS_REAL = _ROWS_PER_CHUNK * _SEQ
_IDS_PAD = 104
_CHUNKS = _B_PER_W // _ROWS_PER_CHUNK

_BM = 512
_MB = -(-30522 // _BM)


def _t1_body(e_ref, w_ref, o_ref):
    o_ref[...] = jnp.dot(e_ref[...], w_ref[...],
                         preferred_element_type=jnp.float32)


def _pool_body(ids_hbm, t1_hbm, out_hbm, idx_v, rows_v, acc_v, sem0, sem1):
    wid = lax.axis_index("s") * _NC + lax.axis_index("c")
    pltpu.sync_copy(ids_hbm.at[wid], idx_v)
    sems = (sem0, sem1)

    for b in range(2):
        pltpu.async_copy(t1_hbm.at[idx_v.at[b]], rows_v.at[b], sems[b])

    def pair_body(c2, carry):
        for b in range(2):
            c = 2 * c2 + b
            pltpu.make_async_copy(t1_hbm.at[idx_v.at[c]],
                                  rows_v.at[b], sems[b]).wait()
            for k in range(_ROWS_PER_CHUNK):
                def seq_body(s, acc):
                    r = k * _SEQ + s
                    return tuple(acc[j] + rows_v[b, r, pl.ds(16 * j, 16)]
                                 for j in range(_HV))
                acc = lax.fori_loop(
                    0, _SEQ, seq_body,
                    tuple(jnp.zeros((16,), jnp.float32) for _ in range(_HV)))
                row = c * _ROWS_PER_CHUNK + k
                for j in range(_HV):
                    acc_v[row, pl.ds(16 * j, 16)] = acc[j]

            @pl.when(c + 2 < _CHUNKS)
            def _prefetch():
                pltpu.async_copy(t1_hbm.at[idx_v.at[c + 2]],
                                 rows_v.at[b], sems[b])
        return carry

    lax.fori_loop(0, _CHUNKS // 2, pair_body, 0)
    pltpu.sync_copy(acc_v, out_hbm.at[pl.ds(wid * _B_PER_W, _B_PER_W)])


def _mlp_body(p_ref, b1_ref, w2_ref, b2_ref, o_ref):
    h = jnp.maximum(p_ref[...] * (1.0 / _SEQ) + b1_ref[...], 0.0)
    o_ref[...] = jnp.dot(h, w2_ref[...],
                         preferred_element_type=jnp.float32) + b2_ref[...]


def kernel(input_ids, emb_table, W1, b1, W2, b2):
    T1 = pl.pallas_call(
        _t1_body,
        grid=(_MB,),
        in_specs=[pl.BlockSpec((_BM, _D), lambda i: (i, 0)),
                  pl.BlockSpec((_D, _H), lambda i: (0, 0))],
        out_specs=pl.BlockSpec((_BM, _H), lambda i: (i, 0)),
        out_shape=jax.ShapeDtypeStruct((_MB * _BM, _H), jnp.float32),
    )(emb_table, W1)

    ids = input_ids.astype(jnp.int32).reshape(_NW, _CHUNKS, _IDS_REAL)
    npad = _IDS_PAD - _IDS_REAL
    pad = (jnp.arange(_NW * _CHUNKS * npad, dtype=jnp.int32) % 30522)
    ids = jnp.concatenate([ids, pad.reshape(_NW, _CHUNKS, npad)], axis=2)

    pool = functools.partial(
        pl.kernel,
        mesh=plsc.VectorSubcoreMesh(core_axis_name="c", subcore_axis_name="s"),
        out_type=jax.ShapeDtypeStruct((_BATCH, _H), jnp.float32),
        scratch_types=[
            pltpu.VMEM((_CHUNKS, _IDS_PAD), jnp.int32),
            pltpu.VMEM((2, _IDS_PAD, _H), jnp.float32),
            pltpu.VMEM((_B_PER_W, _H), jnp.float32),
            pltpu.SemaphoreType.DMA,
            pltpu.SemaphoreType.DMA,
        ],
    )(_pool_body)
    P = pool(ids, T1)

    return pl.pallas_call(
        _mlp_body,
        grid=(8,),
        in_specs=[pl.BlockSpec((_BATCH // 8, _H), lambda i: (i, 0)),
                  pl.BlockSpec((1, _H), lambda i: (0, 0)),
                  pl.BlockSpec((_H, _NE), lambda i: (0, 0)),
                  pl.BlockSpec((1, _NE), lambda i: (0, 0))],
        out_specs=pl.BlockSpec((_BATCH // 8, _NE), lambda i: (i, 0)),
        out_shape=jax.ShapeDtypeStruct((_BATCH, _NE), jnp.float32),
    )(P, b1[None], W2, b2[None])


# speedup vs baseline: 2.6409x; 1.0000x over previous
import functools

import jax
import jax.numpy as jnp
from jax import lax
from jax.experimental import pallas as pl
from jax.experimental.pallas import tpu as pltpu
from jax.experimental.pallas import tpu_sc as plsc

_INFO = plsc.get_sparse_core_info()
_NC, _NS, _L = _INFO.num_cores, _INFO.num_subcores, _INFO.num_lanes
_NW = _NC * _NS

_BATCH = 4096
_SEQ = 50
_D = 768
_H = 256
_NE = 28
_HV = _H // 16

_B_PER_W = _BATCH // _NW
_ROWS_PER_CHUNK = 2
_IDS_REAL = _ROWS_PER_CHUNK * _SEQ
_IDS_PAD = 104
_CHUNKS = _B_PER_W // _ROWS_PER_CHUNK

_BM = 512
_MB = -(-30522 // _BM)


def _t1_body(e_ref, w_ref, o_ref):
    o_ref[...] = jnp.dot(e_ref[...].astype(jnp.bfloat16),
                         w_ref[...].astype(jnp.bfloat16),
                         preferred_element_type=jnp.float32)


def _pool_body(ids_hbm, t1_hbm, out_hbm, idx_v, rows_v, acc_v, sem0, sem1):
    wid = lax.axis_index("s") * _NC + lax.axis_index("c")
    pltpu.sync_copy(ids_hbm.at[wid], idx_v)
    sems = (sem0, sem1)

    for b in range(2):
        pltpu.async_copy(t1_hbm.at[idx_v.at[b]], rows_v.at[b], sems[b])

    def pair_body(c2, carry):
        for b in range(2):
            c = 2 * c2 + b
            pltpu.make_async_copy(t1_hbm.at[idx_v.at[c]],
                                  rows_v.at[b], sems[b]).wait()
            for k in range(_ROWS_PER_CHUNK):
                def seq_body(s, acc):
                    r = k * _SEQ + s
                    return tuple(acc[j] + rows_v[b, r, pl.ds(16 * j, 16)]
                                 for j in range(_HV))
                acc = lax.fori_loop(
                    0, _SEQ, seq_body,
                    tuple(jnp.zeros((16,), jnp.float32) for _ in range(_HV)))
                row = c * _ROWS_PER_CHUNK + k
                for j in range(_HV):
                    acc_v[row, pl.ds(16 * j, 16)] = acc[j]

            @pl.when(c + 2 < _CHUNKS)
            def _prefetch():
                pltpu.async_copy(t1_hbm.at[idx_v.at[c + 2]],
                                 rows_v.at[b], sems[b])
        return carry

    lax.fori_loop(0, _CHUNKS // 2, pair_body, 0)
    pltpu.sync_copy(acc_v, out_hbm.at[pl.ds(wid * _B_PER_W, _B_PER_W)])


def _mlp_body(p_ref, b1_ref, w2_ref, b2_ref, o_ref):
    h = jnp.maximum(p_ref[...] * (1.0 / _SEQ) + b1_ref[...], 0.0)
    o_ref[...] = jnp.dot(h, w2_ref[...],
                         preferred_element_type=jnp.float32) + b2_ref[...]


def kernel(input_ids, emb_table, W1, b1, W2, b2):
    T1 = pl.pallas_call(
        _t1_body,
        grid=(_MB,),
        in_specs=[pl.BlockSpec((_BM, _D), lambda i: (i, 0)),
                  pl.BlockSpec((_D, _H), lambda i: (0, 0))],
        out_specs=pl.BlockSpec((_BM, _H), lambda i: (i, 0)),
        out_shape=jax.ShapeDtypeStruct((_MB * _BM, _H), jnp.float32),
    )(emb_table, W1)

    ids = input_ids.astype(jnp.int32).reshape(_NW, _CHUNKS, _IDS_REAL)
    npad = _IDS_PAD - _IDS_REAL
    pad = (jnp.arange(_NW * _CHUNKS * npad, dtype=jnp.int32) % 30522)
    ids = jnp.concatenate([ids, pad.reshape(_NW, _CHUNKS, npad)], axis=2)

    pool = functools.partial(
        pl.kernel,
        mesh=plsc.VectorSubcoreMesh(core_axis_name="c", subcore_axis_name="s"),
        out_type=jax.ShapeDtypeStruct((_BATCH, _H), jnp.float32),
        scratch_types=[
            pltpu.VMEM((_CHUNKS, _IDS_PAD), jnp.int32),
            pltpu.VMEM((2, _IDS_PAD, _H), jnp.float32),
            pltpu.VMEM((_B_PER_W, _H), jnp.float32),
            pltpu.SemaphoreType.DMA,
            pltpu.SemaphoreType.DMA,
        ],
    )(_pool_body)
    P = pool(ids, T1)

    return pl.pallas_call(
        _mlp_body,
        grid=(8,),
        in_specs=[pl.BlockSpec((_BATCH // 8, _H), lambda i: (i, 0)),
                  pl.BlockSpec((1, _H), lambda i: (0, 0)),
                  pl.BlockSpec((_H, _NE), lambda i: (0, 0)),
                  pl.BlockSpec((1, _NE), lambda i: (0, 0))],
        out_specs=pl.BlockSpec((_BATCH // 8, _NE), lambda i: (i, 0)),
        out_shape=jax.ShapeDtypeStruct((_BATCH, _NE), jnp.float32),
    )(P, b1[None], W2, b2[None])


# speedup vs baseline: 3.1712x; 1.2008x over previous
import functools

import jax
import jax.numpy as jnp
from jax import lax
from jax.experimental import pallas as pl
from jax.experimental.pallas import tpu as pltpu
from jax.experimental.pallas import tpu_sc as plsc

_INFO = plsc.get_sparse_core_info()
_NC, _NS, _L = _INFO.num_cores, _INFO.num_subcores, _INFO.num_lanes
_NW = _NC * _NS

_BATCH = 4096
_SEQ = 50
_D = 768
_H = 256
_NE = 28
_HW = _H // 2
_HV = _H // 16
_WV = _HW // 16

_B_PER_W = _BATCH // _NW
_ROWS_PER_CHUNK = 2
_IDS_REAL = _ROWS_PER_CHUNK * _SEQ
_IDS_PAD = 104
_CHUNKS = _B_PER_W // _ROWS_PER_CHUNK

_BM = 512
_MB = -(-30522 // _BM)


def _bf16_bits(x):
    u = lax.bitcast_convert_type(x, jnp.uint32)
    return (u + jnp.uint32(0x7FFF) + ((u >> 16) & jnp.uint32(1))) >> 16


def _t1_body(e_ref, w_ref, o_ref):
    x = jnp.dot(e_ref[...].astype(jnp.bfloat16),
                w_ref[...].astype(jnp.bfloat16),
                preferred_element_type=jnp.float32)
    lo = _bf16_bits(x[:, :_HW])
    hi = _bf16_bits(x[:, _HW:])
    o_ref[...] = lax.bitcast_convert_type(lo | (hi << 16), jnp.int32)


def _pool_body(ids_hbm, t1_hbm, out_hbm, idx_v, rows_v, acc_v, sem0, sem1):
    wid = lax.axis_index("s") * _NC + lax.axis_index("c")
    pltpu.sync_copy(ids_hbm.at[wid], idx_v)
    sems = (sem0, sem1)

    for b in range(2):
        pltpu.async_copy(t1_hbm.at[idx_v.at[b]], rows_v.at[b], sems[b])

    def pair_body(c2, carry):
        for b in range(2):
            c = 2 * c2 + b
            pltpu.make_async_copy(t1_hbm.at[idx_v.at[c]],
                                  rows_v.at[b], sems[b]).wait()
            for k in range(_ROWS_PER_CHUNK):
                mask_hi = jnp.full((16,), -65536, jnp.int32)
                sh16 = jnp.full((16,), 16, jnp.int32)

                def seq_body(s, acc):
                    r = k * _SEQ + s
                    out = list(acc)
                    for j in range(_WV):
                        w = rows_v[b, r, pl.ds(16 * j, 16)]
                        lo = lax.bitcast_convert_type(w << sh16, jnp.float32)
                        hi = lax.bitcast_convert_type(w & mask_hi, jnp.float32)
                        out[j] = out[j] + lo
                        out[_WV + j] = out[_WV + j] + hi
                    return tuple(out)
                acc = lax.fori_loop(
                    0, _SEQ, seq_body,
                    tuple(jnp.zeros((16,), jnp.float32) for _ in range(_HV)))
                row = c * _ROWS_PER_CHUNK + k
                for j in range(_HV):
                    acc_v[row, pl.ds(16 * j, 16)] = acc[j]

            @pl.when(c + 2 < _CHUNKS)
            def _prefetch():
                pltpu.async_copy(t1_hbm.at[idx_v.at[c + 2]],
                                 rows_v.at[b], sems[b])
        return carry

    lax.fori_loop(0, _CHUNKS // 2, pair_body, 0)
    pltpu.sync_copy(acc_v, out_hbm.at[pl.ds(wid * _B_PER_W, _B_PER_W)])


def _mlp_body(p_ref, b1_ref, w2_ref, b2_ref, o_ref):
    h = jnp.maximum(p_ref[...] * (1.0 / _SEQ) + b1_ref[...], 0.0)
    o_ref[...] = jnp.dot(h, w2_ref[...],
                         preferred_element_type=jnp.float32) + b2_ref[...]


def kernel(input_ids, emb_table, W1, b1, W2, b2):
    T1 = pl.pallas_call(
        _t1_body,
        grid=(_MB,),
        in_specs=[pl.BlockSpec((_BM, _D), lambda i: (i, 0)),
                  pl.BlockSpec((_D, _H), lambda i: (0, 0))],
        out_specs=pl.BlockSpec((_BM, _HW), lambda i: (i, 0)),
        out_shape=jax.ShapeDtypeStruct((_MB * _BM, _HW), jnp.int32),
    )(emb_table, W1)

    ids = input_ids.astype(jnp.int32).reshape(_NW, _CHUNKS, _IDS_REAL)
    npad = _IDS_PAD - _IDS_REAL
    pad = (jnp.arange(_NW * _CHUNKS * npad, dtype=jnp.int32) % 30522)
    ids = jnp.concatenate([ids, pad.reshape(_NW, _CHUNKS, npad)], axis=2)

    pool = functools.partial(
        pl.kernel,
        mesh=plsc.VectorSubcoreMesh(core_axis_name="c", subcore_axis_name="s"),
        out_type=jax.ShapeDtypeStruct((_BATCH, _H), jnp.float32),
        scratch_types=[
            pltpu.VMEM((_CHUNKS, _IDS_PAD), jnp.int32),
            pltpu.VMEM((2, _IDS_PAD, _HW), jnp.int32),
            pltpu.VMEM((_B_PER_W, _H), jnp.float32),
            pltpu.SemaphoreType.DMA,
            pltpu.SemaphoreType.DMA,
        ],
    )(_pool_body)
    P = pool(ids, T1)

    return pl.pallas_call(
        _mlp_body,
        grid=(8,),
        in_specs=[pl.BlockSpec((_BATCH // 8, _H), lambda i: (i, 0)),
                  pl.BlockSpec((1, _H), lambda i: (0, 0)),
                  pl.BlockSpec((_H, _NE), lambda i: (0, 0)),
                  pl.BlockSpec((1, _NE), lambda i: (0, 0))],
        out_specs=pl.BlockSpec((_BATCH // 8, _NE), lambda i: (i, 0)),
        out_shape=jax.ShapeDtypeStruct((_BATCH, _NE), jnp.float32),
    )(P, b1[None], W2, b2[None])


# speedup vs baseline: 4.2193x; 1.3305x over previous
import functools

import jax
import jax.numpy as jnp
from jax import lax
from jax.experimental import pallas as pl
from jax.experimental.pallas import tpu as pltpu
from jax.experimental.pallas import tpu_sc as plsc

_INFO = plsc.get_sparse_core_info()
_NC, _NS, _L = _INFO.num_cores, _INFO.num_subcores, _INFO.num_lanes
_NW = _NC * _NS

_BATCH = 4096
_SEQ = 50
_D = 768
_H = 256
_NE = 28
_HW = _H // 2
_HV = _H // 16
_WV = _HW // 16

_B_PER_W = _BATCH // _NW
_ROWS_PER_CHUNK = 8
_GPC = 4
_IDS_REAL = 2 * _SEQ
_IDS_PAD = 104
_CHUNKS = _B_PER_W // _ROWS_PER_CHUNK

_BM = 3840
_MB = -(-30522 // _BM)


def _bf16_bits(x):
    u = lax.bitcast_convert_type(x, jnp.uint32)
    return (u + jnp.uint32(0x7FFF) + ((u >> 16) & jnp.uint32(1))) >> 16


def _t1_body(e_ref, w_ref, o_ref):
    x = jnp.dot(e_ref[...].astype(jnp.bfloat16),
                w_ref[...].astype(jnp.bfloat16),
                preferred_element_type=jnp.float32)
    lo = _bf16_bits(x[:, :_HW])
    hi = _bf16_bits(x[:, _HW:])
    o_ref[...] = lax.bitcast_convert_type(lo | (hi << 16), jnp.int32)


def _pool_body(ids_hbm, t1_hbm, out_hbm, idx_v, rows_v, stage_v,
               sem0, sem1, osem0, osem1):
    wid = lax.axis_index("s") * _NC + lax.axis_index("c")
    base = wid * _B_PER_W
    pltpu.sync_copy(ids_hbm.at[wid], idx_v)
    sems = (sem0, sem1)
    osems = (osem0, osem1)

    for b in range(2):
        for g in range(_GPC):
            pltpu.async_copy(t1_hbm.at[idx_v.at[b, g]],
                             rows_v.at[b, g], sems[b])

    def pair_body(c2, carry):
        for b in range(2):
            c = 2 * c2 + b
            for g in range(_GPC):
                pltpu.make_async_copy(t1_hbm.at[idx_v.at[c, g]],
                                      rows_v.at[b, g], sems[b]).wait()
            @pl.when(c2 >= 1)
            def _drain_out():
                pltpu.make_async_copy(
                    stage_v.at[b],
                    out_hbm.at[pl.ds(base, _ROWS_PER_CHUNK)],
                    osems[b]).wait()

            for k in range(_ROWS_PER_CHUNK):
                mask_hi = jnp.full((16,), -65536, jnp.int32)
                sh16 = jnp.full((16,), 16, jnp.int32)

                def seq_body(s, acc):
                    out = list(acc)
                    for u in range(5):
                        r = (k % 2) * _SEQ + s * 5 + u
                        for j in range(_WV):
                            w = rows_v[b, k // 2, r, pl.ds(16 * j, 16)]
                            lo = lax.bitcast_convert_type(w << sh16,
                                                          jnp.float32)
                            hi = lax.bitcast_convert_type(w & mask_hi,
                                                          jnp.float32)
                            out[j] = out[j] + lo
                            out[_WV + j] = out[_WV + j] + hi
                    return tuple(out)
                acc = lax.fori_loop(
                    0, _SEQ // 5, seq_body,
                    tuple(jnp.zeros((16,), jnp.float32) for _ in range(_HV)))
                for j in range(_HV):
                    stage_v[b, k, pl.ds(16 * j, 16)] = acc[j]

            pltpu.async_copy(
                stage_v.at[b],
                out_hbm.at[pl.ds(base + c * _ROWS_PER_CHUNK,
                                 _ROWS_PER_CHUNK)],
                osems[b])

            @pl.when(c + 2 < _CHUNKS)
            def _prefetch():
                for g in range(_GPC):
                    pltpu.async_copy(t1_hbm.at[idx_v.at[c + 2, g]],
                                     rows_v.at[b, g], sems[b])
        return carry

    lax.fori_loop(0, _CHUNKS // 2, pair_body, 0)
    for b in range(2):
        pltpu.make_async_copy(
            stage_v.at[b],
            out_hbm.at[pl.ds(base, _ROWS_PER_CHUNK)],
            osems[b]).wait()


def _mlp_body(p_ref, b1_ref, w2_ref, b2_ref, o_ref):
    h = jnp.maximum(p_ref[...] * (1.0 / _SEQ) + b1_ref[...], 0.0)
    o_ref[...] = jnp.dot(h, w2_ref[...],
                         preferred_element_type=jnp.float32) + b2_ref[...]


def kernel(input_ids, emb_table, W1, b1, W2, b2):
    T1 = pl.pallas_call(
        _t1_body,
        grid=(_MB,),
        in_specs=[pl.BlockSpec((_BM, _D), lambda i: (i, 0)),
                  pl.BlockSpec((_D, _H), lambda i: (0, 0))],
        out_specs=pl.BlockSpec((_BM, _HW), lambda i: (i, 0)),
        out_shape=jax.ShapeDtypeStruct((_MB * _BM, _HW), jnp.int32),
    )(emb_table, W1)

    ids = input_ids.astype(jnp.int32).reshape(_NW, _CHUNKS, _GPC, _IDS_REAL)
    npad = _IDS_PAD - _IDS_REAL
    pad = (jnp.arange(_NW * _CHUNKS * _GPC * npad, dtype=jnp.int32) % 30522)
    ids = jnp.concatenate(
        [ids, pad.reshape(_NW, _CHUNKS, _GPC, npad)], axis=3)

    pool = functools.partial(
        pl.kernel,
        mesh=plsc.VectorSubcoreMesh(core_axis_name="c", subcore_axis_name="s"),
        out_type=jax.ShapeDtypeStruct((_BATCH, _H), jnp.float32),
        scratch_types=[
            pltpu.VMEM((_CHUNKS, _GPC, _IDS_PAD), jnp.int32),
            pltpu.VMEM((2, _GPC, _IDS_PAD, _HW), jnp.int32),
            pltpu.VMEM((2, _ROWS_PER_CHUNK, _H), jnp.float32),
            pltpu.SemaphoreType.DMA,
            pltpu.SemaphoreType.DMA,
            pltpu.SemaphoreType.DMA,
            pltpu.SemaphoreType.DMA,
        ],
    )(_pool_body)
    P = pool(ids, T1)

    return pl.pallas_call(
        _mlp_body,
        grid=(4,),
        in_specs=[pl.BlockSpec((_BATCH // 4, _H), lambda i: (i, 0)),
                  pl.BlockSpec((1, _H), lambda i: (0, 0)),
                  pl.BlockSpec((_H, _NE), lambda i: (0, 0)),
                  pl.BlockSpec((1, _NE), lambda i: (0, 0))],
        out_specs=pl.BlockSpec((_BATCH // 4, _NE), lambda i: (i, 0)),
        out_shape=jax.ShapeDtypeStruct((_BATCH, _NE), jnp.float32),
    )(P, b1[None], W2, b2[None])
